# baseline (speedup 1.0000x reference)
import functools

import jax
import jax.numpy as jnp
from jax import lax
from jax.experimental import pallas as pl
from jax.experimental.pallas import tpu as pltpu
from jax.experimental.pallas import tpu_sc as plsc

N_NODES = 10000
N_EDGES = 320000
IN_CH = 128
HID = 64

NC = 2
NS = 16
NW = NC * NS
LANES = 16

N_PAD = 10240

GCHUNK = 128
NCHUNKS = N_EDGES // GCHUNK

CPW = HID // NW
SCHUNK = 2560
NSCHUNKS = N_EDGES // SCHUNK


def _node_mm_body(x_ref, w_ref, b_ref, out_ref):
    r = jnp.dot(x_ref[...], w_ref[...], preferred_element_type=jnp.float32, precision=lax.Precision.HIGHEST)
    out_ref[...] = r + b_ref[...]


def _node_mm(x, wcat, bcat):
    n, k = x.shape
    blk = 2000
    return pl.pallas_call(
        _node_mm_body,
        grid=(n // blk,),
        in_specs=[
            pl.BlockSpec((blk, k), lambda i: (i, 0)),
            pl.BlockSpec((k, 2 * HID), lambda i: (0, 0)),
            pl.BlockSpec((1, 2 * HID), lambda i: (0, 0)),
        ],
        out_specs=pl.BlockSpec((blk, 2 * HID), lambda i: (i, 0)),
        out_shape=jax.ShapeDtypeStruct((n, 2 * HID), jnp.float32),
    )(x, wcat, bcat)


def _edge_mm_body(pre_ref, w_ref, b_ref, out_ref):
    a = jnp.maximum(pre_ref[...], 0.0)
    r = lax.dot_general(w_ref[...], a, (((0,), (0,)), ((), ())),
                        preferred_element_type=jnp.float32,
                        precision=lax.Precision.HIGHEST)
    out_ref[...] = r + b_ref[...]


def _edge_mm(pre_t, w, bcol):
    e = N_EDGES
    blk = 6400
    return pl.pallas_call(
        _edge_mm_body,
        grid=(e // blk,),
        in_specs=[
            pl.BlockSpec((HID, blk), lambda i: (0, i)),
            pl.BlockSpec((HID, HID), lambda i: (0, 0)),
            pl.BlockSpec((HID, 1), lambda i: (0, 0)),
        ],
        out_specs=pl.BlockSpec((HID, blk), lambda i: (0, i)),
        out_shape=jax.ShapeDtypeStruct((HID, e), jnp.float32),
    )(pre_t, w, bcol)


def _t_mm_body(lhs_ref, w_ref, b_ref, out_ref):
    r = lax.dot_general(lhs_ref[...], w_ref[...], (((0,), (0,)), ((), ())),
                        preferred_element_type=jnp.float32,
                        precision=lax.Precision.HIGHEST)
    out_ref[...] = r + b_ref[...]


def _t_mm(lhs_t, wcat, bcat):
    n = lhs_t.shape[1]
    return pl.pallas_call(
        _t_mm_body,
        grid=(1,),
        in_specs=[
            pl.BlockSpec((HID, n), lambda i: (0, 0)),
            pl.BlockSpec((HID, 2 * HID), lambda i: (0, 0)),
            pl.BlockSpec((1, 2 * HID), lambda i: (0, 0)),
        ],
        out_specs=pl.BlockSpec((n, 2 * HID), lambda i: (0, 0)),
        out_shape=jax.ShapeDtypeStruct((n, 2 * HID), jnp.float32),
    )(lhs_t, wcat, bcat)


def _head_mm_body(wt_ref, lhs_ref, b_ref, out_ref):
    r = jnp.dot(wt_ref[...], lhs_ref[...], preferred_element_type=jnp.float32, precision=lax.Precision.HIGHEST)
    out_ref[...] = r + b_ref[...]


def _head_mm(lhs_t, wl_t, bl):
    n = lhs_t.shape[1]
    return pl.pallas_call(
        _head_mm_body,
        grid=(1,),
        in_specs=[
            pl.BlockSpec((1, HID), lambda i: (0, 0)),
            pl.BlockSpec((HID, n), lambda i: (0, 0)),
            pl.BlockSpec((1, 1), lambda i: (0, 0)),
        ],
        out_specs=pl.BlockSpec((1, n), lambda i: (0, 0)),
        out_shape=jax.ShapeDtypeStruct((1, n), jnp.float32),
    )(wl_t, lhs_t, bl)


def _sc_mesh():
    return plsc.VectorSubcoreMesh(
        core_axis_name="c", subcore_axis_name="s",
        num_cores=NC, num_subcores=NS)


GCHW = 80
NCHUNKS_PAD = NW * GCHW
E_PAD = NCHUNKS_PAD * GCHUNK


def _edge_gather_body(c_hbm, dst_hbm, src_hbm, out_hbm,
                      idxd_v, idxs_v, bufd_v, bufs_v, outb_v,
                      semd0, semd1, sems0, sems1, semw0, semw1):
    wid = lax.axis_index("s") * NC + lax.axis_index("c")
    lane = lax.iota(jnp.int32, LANES)
    semd = (semd0, semd1)
    sems = (sems0, sems1)
    semw = (semw0, semw1)
    start = wid * GCHW

    pltpu.sync_copy(dst_hbm.at[pl.ds(start, GCHW)], idxd_v)
    pltpu.sync_copy(src_hbm.at[pl.ds(start, GCHW)], idxs_v)

    def issue_gather(i, b):
        pltpu.async_copy(c_hbm.at[idxd_v.at[i]], bufd_v.at[b], semd[b])
        pltpu.async_copy(c_hbm.at[idxs_v.at[i]], bufs_v.at[b], sems[b])

    def wait_gather(b):
        pltpu.make_async_copy(c_hbm.at[idxd_v.at[0]], bufd_v.at[b], semd[b]).wait()
        pltpu.make_async_copy(c_hbm.at[idxs_v.at[0]], bufs_v.at[b], sems[b]).wait()

    def wait_wb(b):
        pltpu.make_async_copy(outb_v.at[b],
                              out_hbm.at[:, pl.ds(0, GCHUNK)], semw[b]).wait()

    def compute(b):
        def row(r, _):
            rcol = jnp.full((LANES,), r, jnp.int32)
            for s in range(HID // LANES):
                sl = pl.ds(s * LANES, LANES)
                sh = pl.ds(HID + s * LANES, LANES)
                v = bufd_v[b, r, sl] + bufs_v[b, r, sh]
                plsc.store_scatter(outb_v.at[b], [lane + s * LANES, rcol], v)
            return 0

        lax.fori_loop(0, GCHUNK, row, 0, unroll=2)

    issue_gather(0, 0)

    def group(g, _):
        for b in range(2):
            i = g * 2 + b

            @pl.when(i >= 2)
            def _():
                wait_wb(b)

            wait_gather(b)

            @pl.when(i + 1 < GCHW)
            def _():
                issue_gather(i + 1, 1 - b)

            compute(b)
            base = (start + i) * GCHUNK
            pltpu.async_copy(outb_v.at[b],
                             out_hbm.at[:, pl.ds(base, GCHUNK)], semw[b])
        return 0

    lax.fori_loop(0, GCHW // 2, group, 0)
    wait_wb(0)
    wait_wb(1)


def _edge_gather(c, dst2d, src2d):
    kern = pl.kernel(
        _edge_gather_body,
        out_type=jax.ShapeDtypeStruct((HID, E_PAD), jnp.float32),
        mesh=_sc_mesh(),
        compiler_params=pltpu.CompilerParams(needs_layout_passes=False),
        scratch_types=[
            pltpu.VMEM((GCHW, GCHUNK), jnp.int32),
            pltpu.VMEM((GCHW, GCHUNK), jnp.int32),
            pltpu.VMEM((2, GCHUNK, 2 * HID), jnp.float32),
            pltpu.VMEM((2, GCHUNK, 2 * HID), jnp.float32),
            pltpu.VMEM((2, HID, GCHUNK), jnp.float32),
            pltpu.SemaphoreType.DMA,
            pltpu.SemaphoreType.DMA,
            pltpu.SemaphoreType.DMA,
            pltpu.SemaphoreType.DMA,
            pltpu.SemaphoreType.DMA,
            pltpu.SemaphoreType.DMA,
        ],
    )
    return kern(c, dst2d, src2d)


_SPILL_CAP_C = SCHUNK


def _seg_max_body(ht_hbm, dst_hbm, out_hbm,
                  acc0_v, acc1_v, dstb_v, hb_v,
                  spd0_v, spv0_v, spd1_v, spv1_v,
                  semd0, semd1, semh0, semh1):
    wid = lax.axis_index("s") * NC + lax.axis_index("c")
    c0 = wid * CPW
    lane = lax.iota(jnp.int32, LANES)
    semd = (semd0, semd1)
    semh = (semh0, semh1)
    accs = (acc0_v, acc1_v)
    spds = (spd0_v, spd1_v)
    spvs = (spv0_v, spv1_v)

    def zero(i, _):
        zf = jnp.zeros((LANES,), jnp.float32)
        acc0_v[pl.ds(i * LANES, LANES)] = zf
        acc1_v[pl.ds(i * LANES, LANES)] = zf
        return 0

    lax.fori_loop(0, N_PAD // LANES, zero, 0)

    def zsp(i, _):
        zi = jnp.zeros((LANES,), jnp.int32)
        zf = jnp.zeros((LANES,), jnp.float32)
        spd0_v[pl.ds(i * LANES, LANES)] = zi
        spv0_v[pl.ds(i * LANES, LANES)] = zf
        spd1_v[pl.ds(i * LANES, LANES)] = zi
        spv1_v[pl.ds(i * LANES, LANES)] = zf
        return 0

    lax.fori_loop(0, _SPILL_CAP_C // LANES, zsp, 0)

    def issue_loads(i, b):
        base = i * SCHUNK
        pltpu.async_copy(dst_hbm.at[pl.ds(base, SCHUNK)], dstb_v.at[b], semd[b])
        pltpu.async_copy(ht_hbm.at[pl.ds(c0, CPW), pl.ds(base, SCHUNK)],
                         hb_v.at[b], semh[b])

    def wait_loads(b):
        pltpu.make_async_copy(dst_hbm.at[pl.ds(0, SCHUNK)],
                              dstb_v.at[b], semd[b]).wait()
        pltpu.make_async_copy(ht_hbm.at[pl.ds(c0, CPW), pl.ds(0, SCHUNK)],
                              hb_v.at[b], semh[b]).wait()

    def scan_chunk(b):
        def vec(v, offs):
            dv = dstb_v[b, pl.ds(v * LANES, LANES)]
            new_offs = []
            for c in range(CPW):
                acc_v, spd_v, spv_v, off = accs[c], spds[c], spvs[c], offs[c]
                h = hb_v[b, c, pl.ds(v * LANES, LANES)]
                got = plsc.load_gather(acc_v, [dv])
                m = jnp.maximum(h, got)
                plsc.store_scatter(acc_v, [dv], m)
                got2 = plsc.load_gather(acc_v, [dv])
                lost = m > got2
                pcnt = plsc.all_reduce_population_count(lost)
                plsc.store_scatter(spd_v, [off + lane], dv, mask=lost)
                plsc.store_scatter(spv_v, [off + lane], m, mask=lost)
                new_offs.append(off + jnp.where(pcnt > 0, LANES, 0))
            return tuple(new_offs)

        zi = jnp.zeros((LANES,), jnp.int32)
        offs = lax.fori_loop(0, SCHUNK // LANES, vec, (zi,) * CPW, unroll=2)

        for c in range(CPW):
            acc_v, spd_v, spv_v = accs[c], spds[c], spvs[c]
            n = offs[c][0]

            def replay(j, _):
                sdv = spd_v[pl.ds(j * LANES, LANES)]
                sv = spv_v[pl.ds(j * LANES, LANES)]
                got = plsc.load_gather(acc_v, [sdv])
                act = sv > got

                def cond(a):
                    return jnp.any(a)

                def body(a):
                    plsc.store_scatter(acc_v, [sdv], sv, mask=a)
                    g = plsc.load_gather(acc_v, [sdv])
                    return a & (sv > g)

                lax.while_loop(cond, body, act)
                return 0

            lax.fori_loop(0, n // LANES, replay, 0)

    issue_loads(0, 0)

    def group(g, _):
        for b in range(2):
            i = g * 2 + b
            wait_loads(b)
            issue_loads(i + 1, 1 - b)
            scan_chunk(b)
        return 0

    lax.fori_loop(0, (NSCHUNKS - 1) // 2, group, 0)
    wait_loads((NSCHUNKS - 1) % 2)
    scan_chunk((NSCHUNKS - 1) % 2)

    for c in range(CPW):
        pltpu.sync_copy(accs[c], out_hbm.at[c0 + c])


def _seg_max(ht, dst):
    kern = pl.kernel(
        _seg_max_body,
        out_type=jax.ShapeDtypeStruct((HID, N_PAD), jnp.float32),
        mesh=_sc_mesh(),
        compiler_params=pltpu.CompilerParams(needs_layout_passes=False),
        scratch_types=[
            pltpu.VMEM((N_PAD,), jnp.float32),
            pltpu.VMEM((N_PAD,), jnp.float32),
            pltpu.VMEM((2, SCHUNK), jnp.int32),
            pltpu.VMEM((2, CPW, SCHUNK), jnp.float32),
            pltpu.VMEM((_SPILL_CAP_C,), jnp.int32),
            pltpu.VMEM((_SPILL_CAP_C,), jnp.float32),
            pltpu.VMEM((_SPILL_CAP_C,), jnp.int32),
            pltpu.VMEM((_SPILL_CAP_C,), jnp.float32),
            pltpu.SemaphoreType.DMA,
            pltpu.SemaphoreType.DMA,
            pltpu.SemaphoreType.DMA,
            pltpu.SemaphoreType.DMA,
        ],
    )
    return kern(ht, dst)


def kernel(x, edge_index, W1, b1, W2, b2, W3, b3, W4, b4, Wl, bl):
    src = edge_index[0].astype(jnp.int32)
    dst = edge_index[1].astype(jnp.int32)
    pad = ((0, NCHUNKS_PAD - NCHUNKS), (0, 0))
    src2d = jnp.pad(src.reshape(NCHUNKS, GCHUNK), pad)
    dst2d = jnp.pad(dst.reshape(NCHUNKS, GCHUNK), pad)

    w1cat = jnp.concatenate([W1[:IN_CH] - W1[IN_CH:], W1[IN_CH:]], axis=1)
    b1cat = jnp.concatenate([b1, jnp.zeros_like(b1)])[None, :]
    c1 = _node_mm(x, w1cat, b1cat)
    pre1 = _edge_gather(c1, dst2d, src2d)
    h1t = _seg_max(_edge_mm(pre1, W2, b2[:, None]), dst)

    w3cat = jnp.concatenate([W3[:HID] - W3[HID:], W3[HID:]], axis=1)
    b3cat = jnp.concatenate([b3, jnp.zeros_like(b3)])[None, :]
    c2 = _t_mm(h1t, w3cat, b3cat)
    pre2 = _edge_gather(c2, dst2d, src2d)
    h2t = _seg_max(_edge_mm(pre2, W4, b4[:, None]), dst)

    out = _head_mm(h2t, Wl.T, bl[None, :])
    return out[0, :N_NODES]


# speedup vs baseline: 1.6028x; 1.6028x over previous
import functools

import jax
import jax.numpy as jnp
from jax import lax
from jax.experimental import pallas as pl
from jax.experimental.pallas import tpu as pltpu
from jax.experimental.pallas import tpu_sc as plsc

N_NODES = 10000
N_EDGES = 320000
IN_CH = 128
HID = 64

NC = 2
NS = 16
NW = NC * NS
LANES = 16

N_PAD = 10240

GCHUNK = 128
NCHUNKS = N_EDGES // GCHUNK

CPW = HID // NW
SCHUNK = 2560
NSCHUNKS = N_EDGES // SCHUNK


def _node_mm_body(x_ref, w_ref, b_ref, out_ref):
    r = jnp.dot(x_ref[...], w_ref[...], preferred_element_type=jnp.float32, precision=lax.Precision.HIGHEST)
    out_ref[...] = r + b_ref[...]


def _node_mm(x, wcat, bcat):
    n, k = x.shape
    blk = 2000
    return pl.pallas_call(
        _node_mm_body,
        grid=(n // blk,),
        in_specs=[
            pl.BlockSpec((blk, k), lambda i: (i, 0)),
            pl.BlockSpec((k, 2 * HID), lambda i: (0, 0)),
            pl.BlockSpec((1, 2 * HID), lambda i: (0, 0)),
        ],
        out_specs=pl.BlockSpec((blk, 2 * HID), lambda i: (i, 0)),
        out_shape=jax.ShapeDtypeStruct((n, 2 * HID), jnp.float32),
    )(x, wcat, bcat)


def _edge_mm_body(pre_ref, w_ref, b_ref, out_ref):
    a = jnp.maximum(pre_ref[...], 0.0)
    r = lax.dot_general(w_ref[...], a, (((0,), (0,)), ((), ())),
                        preferred_element_type=jnp.float32,
                        precision=lax.Precision.HIGHEST)
    out_ref[...] = r + b_ref[...]


def _edge_mm(pre_t, w, bcol):
    e = N_EDGES
    blk = 6400
    return pl.pallas_call(
        _edge_mm_body,
        grid=(e // blk,),
        in_specs=[
            pl.BlockSpec((HID, blk), lambda i: (0, i)),
            pl.BlockSpec((HID, HID), lambda i: (0, 0)),
            pl.BlockSpec((HID, 1), lambda i: (0, 0)),
        ],
        out_specs=pl.BlockSpec((HID, blk), lambda i: (0, i)),
        out_shape=jax.ShapeDtypeStruct((HID, e), jnp.float32),
    )(pre_t, w, bcol)


def _t_mm_body(lhs_ref, w_ref, b_ref, out_ref):
    r = lax.dot_general(lhs_ref[...], w_ref[...], (((0,), (0,)), ((), ())),
                        preferred_element_type=jnp.float32,
                        precision=lax.Precision.HIGHEST)
    out_ref[...] = r + b_ref[...]


def _t_mm(lhs_t, wcat, bcat):
    n = lhs_t.shape[1]
    return pl.pallas_call(
        _t_mm_body,
        grid=(1,),
        in_specs=[
            pl.BlockSpec((HID, n), lambda i: (0, 0)),
            pl.BlockSpec((HID, 2 * HID), lambda i: (0, 0)),
            pl.BlockSpec((1, 2 * HID), lambda i: (0, 0)),
        ],
        out_specs=pl.BlockSpec((n, 2 * HID), lambda i: (0, 0)),
        out_shape=jax.ShapeDtypeStruct((n, 2 * HID), jnp.float32),
    )(lhs_t, wcat, bcat)


def _head_mm_body(wt_ref, lhs_ref, b_ref, out_ref):
    r = jnp.dot(wt_ref[...], lhs_ref[...], preferred_element_type=jnp.float32, precision=lax.Precision.HIGHEST)
    out_ref[...] = r + b_ref[...]


def _head_mm(lhs_t, wl_t, bl):
    n = lhs_t.shape[1]
    return pl.pallas_call(
        _head_mm_body,
        grid=(1,),
        in_specs=[
            pl.BlockSpec((1, HID), lambda i: (0, 0)),
            pl.BlockSpec((HID, n), lambda i: (0, 0)),
            pl.BlockSpec((1, 1), lambda i: (0, 0)),
        ],
        out_specs=pl.BlockSpec((1, n), lambda i: (0, 0)),
        out_shape=jax.ShapeDtypeStruct((1, n), jnp.float32),
    )(wl_t, lhs_t, bl)


def _sc_mesh():
    return plsc.VectorSubcoreMesh(
        core_axis_name="c", subcore_axis_name="s",
        num_cores=NC, num_subcores=NS)


GCHW = 80
NCHUNKS_PAD = NW * GCHW
E_PAD = NCHUNKS_PAD * GCHUNK


def _edge_gather_body(c_hbm, dst_hbm, src_hbm, out_hbm,
                      idxd_v, idxs_v, bufd_v, bufs_v, outb_v,
                      semd0, semd1, sems0, sems1, semw0, semw1):
    wid = lax.axis_index("s") * NC + lax.axis_index("c")
    lane = lax.iota(jnp.int32, LANES)
    semd = (semd0, semd1)
    sems = (sems0, sems1)
    semw = (semw0, semw1)
    start = wid * GCHW

    pltpu.sync_copy(dst_hbm.at[pl.ds(start, GCHW)], idxd_v)
    pltpu.sync_copy(src_hbm.at[pl.ds(start, GCHW)], idxs_v)

    def issue_gather(i, b):
        pltpu.async_copy(c_hbm.at[idxd_v.at[i]], bufd_v.at[b], semd[b])
        pltpu.async_copy(c_hbm.at[idxs_v.at[i]], bufs_v.at[b], sems[b])

    def wait_gather(b):
        pltpu.make_async_copy(c_hbm.at[idxd_v.at[0]], bufd_v.at[b], semd[b]).wait()
        pltpu.make_async_copy(c_hbm.at[idxs_v.at[0]], bufs_v.at[b], sems[b]).wait()

    def wait_wb(b):
        pltpu.make_async_copy(outb_v.at[b],
                              out_hbm.at[:, pl.ds(0, GCHUNK)], semw[b]).wait()

    def compute(b):
        def row(r, _):
            rcol = jnp.full((LANES,), r, jnp.int32)
            for s in range(HID // LANES):
                sl = pl.ds(s * LANES, LANES)
                sh = pl.ds(HID + s * LANES, LANES)
                v = bufd_v[b, r, sl] + bufs_v[b, r, sh]
                plsc.store_scatter(outb_v.at[b], [lane + s * LANES, rcol], v)
            return 0

        lax.fori_loop(0, GCHUNK, row, 0, unroll=2)

    issue_gather(0, 0)

    def group(g, _):
        for b in range(2):
            i = g * 2 + b

            @pl.when(i >= 2)
            def _():
                wait_wb(b)

            wait_gather(b)

            @pl.when(i + 1 < GCHW)
            def _():
                issue_gather(i + 1, 1 - b)

            compute(b)
            base = (start + i) * GCHUNK
            pltpu.async_copy(outb_v.at[b],
                             out_hbm.at[:, pl.ds(base, GCHUNK)], semw[b])
        return 0

    lax.fori_loop(0, GCHW // 2, group, 0)
    wait_wb(0)
    wait_wb(1)


def _edge_gather(c, dst2d, src2d):
    kern = pl.kernel(
        _edge_gather_body,
        out_type=jax.ShapeDtypeStruct((HID, E_PAD), jnp.float32),
        mesh=_sc_mesh(),
        compiler_params=pltpu.CompilerParams(needs_layout_passes=False),
        scratch_types=[
            pltpu.VMEM((GCHW, GCHUNK), jnp.int32),
            pltpu.VMEM((GCHW, GCHUNK), jnp.int32),
            pltpu.VMEM((2, GCHUNK, 2 * HID), jnp.float32),
            pltpu.VMEM((2, GCHUNK, 2 * HID), jnp.float32),
            pltpu.VMEM((2, HID, GCHUNK), jnp.float32),
            pltpu.SemaphoreType.DMA,
            pltpu.SemaphoreType.DMA,
            pltpu.SemaphoreType.DMA,
            pltpu.SemaphoreType.DMA,
            pltpu.SemaphoreType.DMA,
            pltpu.SemaphoreType.DMA,
        ],
    )
    return kern(c, dst2d, src2d)


_SPILL_CAP_C = SCHUNK


def _seg_max_body(ht_hbm, dst_hbm, out_hbm,
                  acc0_v, acc1_v, dstb_v, hb_v,
                  spd0_v, spv0_v, spd1_v, spv1_v,
                  semd0, semd1, semh0, semh1):
    wid = lax.axis_index("s") * NC + lax.axis_index("c")
    c0 = wid * CPW
    lane = lax.iota(jnp.int32, LANES)
    semd = (semd0, semd1)
    semh = (semh0, semh1)
    accs = (acc0_v, acc1_v)
    spds = (spd0_v, spd1_v)
    spvs = (spv0_v, spv1_v)

    def zero(i, _):
        zf = jnp.zeros((LANES,), jnp.float32)
        acc0_v[pl.ds(i * LANES, LANES)] = zf
        acc1_v[pl.ds(i * LANES, LANES)] = zf
        return 0

    lax.fori_loop(0, N_PAD // LANES, zero, 0)

    def zsp(i, _):
        zi = jnp.zeros((LANES,), jnp.int32)
        zf = jnp.zeros((LANES,), jnp.float32)
        spd0_v[pl.ds(i * LANES, LANES)] = zi
        spv0_v[pl.ds(i * LANES, LANES)] = zf
        spd1_v[pl.ds(i * LANES, LANES)] = zi
        spv1_v[pl.ds(i * LANES, LANES)] = zf
        return 0

    lax.fori_loop(0, _SPILL_CAP_C // LANES, zsp, 0)

    def issue_loads(i, b):
        base = i * SCHUNK
        pltpu.async_copy(dst_hbm.at[pl.ds(base, SCHUNK)], dstb_v.at[b], semd[b])
        pltpu.async_copy(ht_hbm.at[pl.ds(c0, CPW), pl.ds(base, SCHUNK)],
                         hb_v.at[b], semh[b])

    def wait_loads(b):
        pltpu.make_async_copy(dst_hbm.at[pl.ds(0, SCHUNK)],
                              dstb_v.at[b], semd[b]).wait()
        pltpu.make_async_copy(ht_hbm.at[pl.ds(c0, CPW), pl.ds(0, SCHUNK)],
                              hb_v.at[b], semh[b]).wait()

    def scan_chunk(b):
        def vec(v, offs):
            dv = dstb_v[b, pl.ds(v * LANES, LANES)]
            _, last = plsc.scan_count(dv)
            nlast = plsc.all_reduce_population_count(last)
            extra = ~last
            adv = jnp.where(nlast < LANES, LANES, 0)
            new_offs = []
            for c in range(CPW):
                acc_v, spd_v, spv_v, off = accs[c], spds[c], spvs[c], offs[c]
                h = hb_v[b, c, pl.ds(v * LANES, LANES)]
                got = plsc.load_gather(acc_v, [dv])
                m = jnp.maximum(h, got)
                plsc.store_scatter(acc_v, [dv], m, mask=last)
                plsc.store_scatter(spd_v, [off + lane], dv, mask=extra)
                plsc.store_scatter(spv_v, [off + lane], m, mask=extra)
                new_offs.append(off + adv)
            return tuple(new_offs)

        zi = jnp.zeros((LANES,), jnp.int32)
        offs = lax.fori_loop(0, SCHUNK // LANES, vec, (zi,) * CPW, unroll=2)

        for c in range(CPW):
            acc_v, spd_v, spv_v = accs[c], spds[c], spvs[c]
            n = offs[c][0]

            def replay(j, _):
                sdv = spd_v[pl.ds(j * LANES, LANES)]
                sv = spv_v[pl.ds(j * LANES, LANES)]
                got = plsc.load_gather(acc_v, [sdv])
                act = sv > got

                def cond(a):
                    return jnp.any(a)

                def body(a):
                    plsc.store_scatter(acc_v, [sdv], sv, mask=a)
                    g = plsc.load_gather(acc_v, [sdv])
                    return a & (sv > g)

                lax.while_loop(cond, body, act)
                return 0

            lax.fori_loop(0, n // LANES, replay, 0)

    issue_loads(0, 0)

    def group(g, _):
        for b in range(2):
            i = g * 2 + b
            wait_loads(b)
            issue_loads(i + 1, 1 - b)
            scan_chunk(b)
        return 0

    lax.fori_loop(0, (NSCHUNKS - 1) // 2, group, 0)
    wait_loads((NSCHUNKS - 1) % 2)
    scan_chunk((NSCHUNKS - 1) % 2)

    for c in range(CPW):
        pltpu.sync_copy(accs[c], out_hbm.at[c0 + c])


def _seg_max(ht, dst):
    kern = pl.kernel(
        _seg_max_body,
        out_type=jax.ShapeDtypeStruct((HID, N_PAD), jnp.float32),
        mesh=_sc_mesh(),
        compiler_params=pltpu.CompilerParams(needs_layout_passes=False),
        scratch_types=[
            pltpu.VMEM((N_PAD,), jnp.float32),
            pltpu.VMEM((N_PAD,), jnp.float32),
            pltpu.VMEM((2, SCHUNK), jnp.int32),
            pltpu.VMEM((2, CPW, SCHUNK), jnp.float32),
            pltpu.VMEM((_SPILL_CAP_C,), jnp.int32),
            pltpu.VMEM((_SPILL_CAP_C,), jnp.float32),
            pltpu.VMEM((_SPILL_CAP_C,), jnp.int32),
            pltpu.VMEM((_SPILL_CAP_C,), jnp.float32),
            pltpu.SemaphoreType.DMA,
            pltpu.SemaphoreType.DMA,
            pltpu.SemaphoreType.DMA,
            pltpu.SemaphoreType.DMA,
        ],
    )
    return kern(ht, dst)


def kernel(x, edge_index, W1, b1, W2, b2, W3, b3, W4, b4, Wl, bl):
    src = edge_index[0].astype(jnp.int32)
    dst = edge_index[1].astype(jnp.int32)
    fill = (jnp.arange((NCHUNKS_PAD - NCHUNKS) * GCHUNK, dtype=jnp.int32)
            % N_NODES).reshape(NCHUNKS_PAD - NCHUNKS, GCHUNK)
    src2d = jnp.concatenate([src.reshape(NCHUNKS, GCHUNK), fill])
    dst2d = jnp.concatenate([dst.reshape(NCHUNKS, GCHUNK), fill])

    w1cat = jnp.concatenate([W1[:IN_CH] - W1[IN_CH:], W1[IN_CH:]], axis=1)
    b1cat = jnp.concatenate([b1, jnp.zeros_like(b1)])[None, :]
    c1 = _node_mm(x, w1cat, b1cat)
    pre1 = _edge_gather(c1, dst2d, src2d)
    h1t = _seg_max(_edge_mm(pre1, W2, b2[:, None]), dst)

    w3cat = jnp.concatenate([W3[:HID] - W3[HID:], W3[HID:]], axis=1)
    b3cat = jnp.concatenate([b3, jnp.zeros_like(b3)])[None, :]
    c2 = _t_mm(h1t, w3cat, b3cat)
    pre2 = _edge_gather(c2, dst2d, src2d)
    h2t = _seg_max(_edge_mm(pre2, W4, b4[:, None]), dst)

    out = _head_mm(h2t, Wl.T, bl[None, :])
    return out[0, :N_NODES]
